# baseline (speedup 1.0000x reference)
import functools

import jax
import jax.numpy as jnp
from jax import lax
from jax.experimental import pallas as pl
from jax.experimental.pallas import tpu as pltpu
from jax.experimental.pallas import tpu_sc as plsc

NFE = 6400000
NCAND = 1000000
NSAMP = 200000
NSEG = 1024

NC, NS, L = 2, 16, 16
NW = NC * NS

UC = NCAND // 64
UW = UC // NW
CW = UW * 64
UC_EXTRA = UC - UW * NW
US = NSAMP // 64
SW = US // NW
SC_N = SW * 64
US_EXTRA = US - SW * NW

_mesh = plsc.VectorSubcoreMesh(
    core_axis_name="c", subcore_axis_name="s",
    num_cores=NC, num_subcores=NS)

_NEG_BIG = -3.0e38

_params = pltpu.CompilerParams(
    needs_layout_passes=False, use_tc_tiling_on_sc=False)


@functools.partial(
    pl.kernel,
    out_type=(
        jax.ShapeDtypeStruct((NCAND,), jnp.float32),
        jax.ShapeDtypeStruct((NC, NSEG), jnp.float32),
        jax.ShapeDtypeStruct((NC, L), jnp.float32),
    ),
    mesh=_mesh,
    compiler_params=_params,
    scratch_types=(
        pltpu.VMEM((CW,), jnp.int32),
        pltpu.VMEM((CW,), jnp.int32),
        pltpu.VMEM((CW,), jnp.float32),
        pltpu.VMEM((CW,), jnp.float32),
        pltpu.VMEM((64,), jnp.int32),
        pltpu.VMEM((64,), jnp.int32),
        pltpu.VMEM((64,), jnp.float32),
        pltpu.VMEM((64,), jnp.float32),
        pltpu.VMEM((L,), jnp.float32),
        pltpu.VMEM((L,), jnp.float32),
        pltpu.VMEM((NS, L), jnp.float32),
        pltpu.VMEM((NSEG,), jnp.float32),
        pltpu.VMEM_SHARED((NS, L), jnp.float32),
        pltpu.VMEM_SHARED((NSEG,), jnp.float32),
        pltpu.SemaphoreType.DMA,
        pltpu.SemaphoreType.DMA,
    ),
)
def _k12(cols_hbm, u_hbm, logits_hbm,
         z_hbm, sums_hbm, scmax_hbm,
         ei_v, eg_v, lg_v, u_v, ex_ei, ex_eg, ex_lg, ex_u,
         pm_v, max_ref, maxbuf_v, zeros_v, sh_max, sh_sums, sem, sem2):
    cid = lax.axis_index("c")
    sid = lax.axis_index("s")
    wid = sid * NC + cid
    c0 = wid * CW
    max_ref[:] = jnp.full((L,), _NEG_BIG, jnp.float32)

    CH = CW // 2
    pltpu.sync_copy(cols_hbm.at[pl.ds(NCAND + c0, CW)], ei_v)
    ga = pltpu.async_copy(
        logits_hbm.at[ei_v.at[pl.ds(0, CH)]], lg_v.at[pl.ds(0, CH)], sem)
    gb = pltpu.async_copy(
        logits_hbm.at[ei_v.at[pl.ds(CH, CH)]], lg_v.at[pl.ds(CH, CH)], sem2)
    pltpu.sync_copy(u_hbm.at[pl.ds(c0, CW)], u_v)
    pltpu.sync_copy(cols_hbm.at[pl.ds(c0, CW)], eg_v)

    for t in range(NSEG // L):
        zeros_v[pl.ds(t * L, L)] = jnp.zeros((L,), jnp.float32)

    @pl.when(sid == 0)
    def _init():
        pltpu.sync_copy(zeros_v, sh_sums)

    @pl.when(wid < UC_EXTRA)
    def _extra_z():
        e0 = (UW * NW + wid) * 64
        pltpu.sync_copy(cols_hbm.at[pl.ds(NCAND + e0, 64)], ex_ei)
        pltpu.sync_copy(u_hbm.at[pl.ds(e0, 64)], ex_u)
        pltpu.sync_copy(cols_hbm.at[pl.ds(e0, 64)], ex_eg)
        pltpu.sync_copy(logits_hbm.at[ex_ei], ex_lg)
        for j in range(4):
            sl = pl.ds(j * L, L)
            zj = ex_lg[sl] + ex_u[sl]
            ex_lg[sl] = zj
            max_ref[:] = jnp.maximum(max_ref[:], zj)
        pltpu.sync_copy(ex_lg, z_hbm.at[pl.ds(e0, 64)])

    def zrow(r, mx):
        for cj in range(16):
            sl = pl.ds(r * 256 + cj * L, L)
            zj = lg_v[sl] + u_v[sl]
            lg_v[sl] = zj
            mx = jnp.maximum(mx, zj)
        return mx
    ga.wait()
    mx = lax.fori_loop(0, CH // 256, zrow, max_ref[:])
    gb.wait()
    mx = lax.fori_loop(CH // 256, CW // 256, zrow, mx)
    pm_v[:] = mx
    zw = pltpu.async_copy(lg_v, z_hbm.at[pl.ds(c0, CW)], sem)

    pltpu.sync_copy(pm_v, sh_max.at[sid])
    plsc.subcore_barrier()
    pltpu.sync_copy(sh_max, maxbuf_v)

    def mrow(r, acc):
        return jnp.maximum(acc, maxbuf_v[r, :])
    m16 = lax.fori_loop(0, NS, mrow, jnp.full((L,), _NEG_BIG, jnp.float32))
    m = jnp.max(m16)
    zw.wait()

    def erow(r, _):
        for cj in range(16):
            sl = pl.ds(r * 256 + cj * L, L)
            lg_v[sl] = jnp.exp(lg_v[sl] - m)
        return 0
    lax.fori_loop(0, CW // 256, erow, 0)

    pltpu.async_copy(lg_v, sh_sums.at[eg_v], sem, add=True).wait()

    @pl.when(wid < UC_EXTRA)
    def _extra_sum():
        for j in range(4):
            sl = pl.ds(j * L, L)
            ex_lg[sl] = jnp.exp(ex_lg[sl] - m)
        pltpu.sync_copy(ex_lg, sh_sums.at[ex_eg], add=True)

    plsc.subcore_barrier()

    @pl.when(sid == 0)
    def _out():
        pltpu.sync_copy(sh_sums, sums_hbm.at[cid])
        pm_v[:] = jnp.zeros((L,), jnp.float32) + m
        pltpu.sync_copy(pm_v, scmax_hbm.at[cid])


@functools.partial(
    pl.kernel,
    out_type=jax.ShapeDtypeStruct((NSAMP,), jnp.float32),
    mesh=_mesh,
    compiler_params=_params,
    scratch_types=(
        pltpu.VMEM((NC, L), jnp.float32),
        pltpu.VMEM((NC, NSEG), jnp.float32),
        pltpu.VMEM((NSEG,), jnp.float32),
        pltpu.VMEM((SC_N,), jnp.int32),
        pltpu.VMEM((SC_N,), jnp.float32),
        pltpu.VMEM((SC_N,), jnp.int32),
        pltpu.VMEM((SC_N,), jnp.float32),
        pltpu.VMEM((64,), jnp.int32),
        pltpu.VMEM((64,), jnp.float32),
        pltpu.VMEM((64,), jnp.int32),
        pltpu.VMEM((64,), jnp.float32),
        pltpu.SemaphoreType.DMA,
        pltpu.SemaphoreType.DMA,
    ),
)
def _k3(ca_hbm, zf_hbm, egf_hbm, scmax_hbm, sums_hbm, out_hbm,
        mx_v, su_v, stab_v, ca_v, zc_v, egc_v, o_v,
        ex_ca, ex_z, ex_eg, ex_o, sem, sem2):
    wid = lax.axis_index("s") * NC + lax.axis_index("c")
    s0 = wid * SC_N
    pltpu.sync_copy(ca_hbm.at[pl.ds(s0, SC_N)], ca_v)
    g1 = pltpu.async_copy(zf_hbm.at[ca_v], zc_v, sem)
    g2 = pltpu.async_copy(egf_hbm.at[ca_v], egc_v, sem2)

    pltpu.sync_copy(scmax_hbm, mx_v)
    m0 = jnp.max(mx_v[0, :])
    m1 = jnp.max(mx_v[1, :])
    m = jnp.maximum(m0, m1)
    f0 = jnp.exp(jnp.zeros((L,), jnp.float32) + (m0 - m))
    f1 = jnp.exp(jnp.zeros((L,), jnp.float32) + (m1 - m))
    pltpu.sync_copy(sums_hbm, su_v)
    for t in range(NSEG // L):
        sl = pl.ds(t * L, L)
        stab_v[sl] = su_v[0, sl] * f0 + su_v[1, sl] * f1

    g1.wait()
    g2.wait()

    def crow(r, _):
        for cj in range(4):
            sl = pl.ds(r * 64 + cj * L, L)
            s16 = plsc.load_gather(stab_v, [egc_v[sl]])
            y = jnp.exp(zc_v[sl] - m) / s16
            o_v[sl] = (1.0 - y) + y
        return 0
    lax.fori_loop(0, SW, crow, 0)
    pltpu.sync_copy(o_v, out_hbm.at[pl.ds(s0, SC_N)])

    @pl.when(wid < US_EXTRA)
    def _extra():
        e0 = (SW * NW + wid) * 64
        pltpu.sync_copy(ca_hbm.at[pl.ds(e0, 64)], ex_ca)
        pltpu.sync_copy(zf_hbm.at[ex_ca], ex_z)
        pltpu.sync_copy(egf_hbm.at[ex_ca], ex_eg)
        for j in range(4):
            sl = pl.ds(j * L, L)
            s16 = plsc.load_gather(stab_v, [ex_eg[sl]])
            y = jnp.exp(ex_z[sl] - m) / s16
            ex_o[sl] = (1.0 - y) + y
        pltpu.sync_copy(ex_o, out_hbm.at[pl.ds(e0, 64)])


def kernel(candidate_edges, loglog_u, sampled_edges, edges_logits):
    eg = candidate_edges[:, 0]
    cols = jnp.concatenate([eg, candidate_edges[:, 1]])
    ca = sampled_edges[:, 5]
    z, sums, scmax = _k12(cols, loglog_u, edges_logits)
    return _k3(ca, z, eg, scmax, sums)


# speedup vs baseline: 1.0011x; 1.0011x over previous
import functools

import jax
import jax.numpy as jnp
from jax import lax
from jax.experimental import pallas as pl
from jax.experimental.pallas import tpu as pltpu
from jax.experimental.pallas import tpu_sc as plsc

NFE = 6400000
NCAND = 1000000
NSAMP = 200000
NSEG = 1024

NC, NS, L = 2, 16, 16
NW = NC * NS

UC = NCAND // 64
UW = UC // NW
CW = UW * 64
UC_EXTRA = UC - UW * NW
US = NSAMP // 64
SW = US // NW
SC_N = SW * 64
US_EXTRA = US - SW * NW

_mesh = plsc.VectorSubcoreMesh(
    core_axis_name="c", subcore_axis_name="s",
    num_cores=NC, num_subcores=NS)

_NEG_BIG = -3.0e38

_params = pltpu.CompilerParams(
    needs_layout_passes=False, use_tc_tiling_on_sc=False)


@functools.partial(
    pl.kernel,
    out_type=(
        jax.ShapeDtypeStruct((NCAND,), jnp.float32),
        jax.ShapeDtypeStruct((NC, NSEG), jnp.float32),
        jax.ShapeDtypeStruct((NC, L), jnp.float32),
    ),
    mesh=_mesh,
    compiler_params=_params,
    scratch_types=(
        pltpu.VMEM((CW,), jnp.int32),
        pltpu.VMEM((CW,), jnp.int32),
        pltpu.VMEM((CW,), jnp.float32),
        pltpu.VMEM((CW,), jnp.float32),
        pltpu.VMEM((64,), jnp.int32),
        pltpu.VMEM((64,), jnp.int32),
        pltpu.VMEM((64,), jnp.float32),
        pltpu.VMEM((64,), jnp.float32),
        pltpu.VMEM((L,), jnp.float32),
        pltpu.VMEM((L,), jnp.float32),
        pltpu.VMEM((NS, L), jnp.float32),
        pltpu.VMEM((NSEG,), jnp.float32),
        pltpu.VMEM_SHARED((NS, L), jnp.float32),
        pltpu.VMEM_SHARED((NSEG,), jnp.float32),
        pltpu.SemaphoreType.DMA,
        pltpu.SemaphoreType.DMA,
    ),
)
def _k12(cols_hbm, u_hbm, logits_hbm,
         z_hbm, sums_hbm, scmax_hbm,
         ei_v, eg_v, lg_v, u_v, ex_ei, ex_eg, ex_lg, ex_u,
         pm_v, max_ref, maxbuf_v, zeros_v, sh_max, sh_sums, sem, sem2):
    cid = lax.axis_index("c")
    sid = lax.axis_index("s")
    wid = sid * NC + cid
    c0 = wid * CW
    max_ref[:] = jnp.full((L,), _NEG_BIG, jnp.float32)

    CH = CW // 2
    pltpu.sync_copy(cols_hbm.at[pl.ds(NCAND + c0, CW)], ei_v)
    ga = pltpu.async_copy(
        logits_hbm.at[ei_v.at[pl.ds(0, CH)]], lg_v.at[pl.ds(0, CH)], sem)
    gb = pltpu.async_copy(
        logits_hbm.at[ei_v.at[pl.ds(CH, CH)]], lg_v.at[pl.ds(CH, CH)], sem2)
    pltpu.sync_copy(u_hbm.at[pl.ds(c0, CW)], u_v)
    pltpu.sync_copy(cols_hbm.at[pl.ds(c0, CW)], eg_v)

    for t in range(NSEG // L):
        zeros_v[pl.ds(t * L, L)] = jnp.zeros((L,), jnp.float32)

    @pl.when(sid == 0)
    def _init():
        pltpu.sync_copy(zeros_v, sh_sums)

    @pl.when(wid < UC_EXTRA)
    def _extra_z():
        e0 = (UW * NW + wid) * 64
        pltpu.sync_copy(cols_hbm.at[pl.ds(NCAND + e0, 64)], ex_ei)
        pltpu.sync_copy(u_hbm.at[pl.ds(e0, 64)], ex_u)
        pltpu.sync_copy(cols_hbm.at[pl.ds(e0, 64)], ex_eg)
        pltpu.sync_copy(logits_hbm.at[ex_ei], ex_lg)
        for j in range(4):
            sl = pl.ds(j * L, L)
            zj = ex_lg[sl] + ex_u[sl]
            ex_lg[sl] = zj
            max_ref[:] = jnp.maximum(max_ref[:], zj)
        pltpu.sync_copy(ex_lg, z_hbm.at[pl.ds(e0, 64)])

    def zrow(r, mx):
        for cj in range(16):
            sl = pl.ds(r * 256 + cj * L, L)
            zj = lg_v[sl] + u_v[sl]
            lg_v[sl] = zj
            mx = jnp.maximum(mx, zj)
        return mx
    ga.wait()
    mx = lax.fori_loop(0, CH // 256, zrow, max_ref[:])
    gb.wait()
    mx = lax.fori_loop(CH // 256, CW // 256, zrow, mx)
    pm_v[:] = mx
    zw = pltpu.async_copy(lg_v, z_hbm.at[pl.ds(c0, CW)], sem)

    pltpu.sync_copy(pm_v, sh_max.at[sid])
    plsc.subcore_barrier()
    pltpu.sync_copy(sh_max, maxbuf_v)

    def mrow(r, acc):
        return jnp.maximum(acc, maxbuf_v[r, :])
    m16 = lax.fori_loop(0, NS, mrow, jnp.full((L,), _NEG_BIG, jnp.float32))
    m = jnp.max(m16)
    zw.wait()

    def erow(r, _):
        for cj in range(16):
            sl = pl.ds(r * 256 + cj * L, L)
            lg_v[sl] = jnp.exp(lg_v[sl] - m)
        return 0
    lax.fori_loop(0, CW // 256, erow, 0)

    pltpu.async_copy(lg_v, sh_sums.at[eg_v], sem, add=True).wait()

    @pl.when(wid < UC_EXTRA)
    def _extra_sum():
        for j in range(4):
            sl = pl.ds(j * L, L)
            ex_lg[sl] = jnp.exp(ex_lg[sl] - m)
        pltpu.sync_copy(ex_lg, sh_sums.at[ex_eg], add=True)

    plsc.subcore_barrier()

    @pl.when(sid == 0)
    def _out():
        pltpu.sync_copy(sh_sums, sums_hbm.at[cid])
        pm_v[:] = jnp.zeros((L,), jnp.float32) + m
        pltpu.sync_copy(pm_v, scmax_hbm.at[cid])


@functools.partial(
    pl.kernel,
    out_type=jax.ShapeDtypeStruct((NSAMP,), jnp.float32),
    mesh=_mesh,
    compiler_params=_params,
    scratch_types=(
        pltpu.VMEM((NC, L), jnp.float32),
        pltpu.VMEM((NC, NSEG), jnp.float32),
        pltpu.VMEM((NSEG,), jnp.float32),
        pltpu.VMEM((SC_N,), jnp.int32),
        pltpu.VMEM((SC_N,), jnp.float32),
        pltpu.VMEM((SC_N,), jnp.int32),
        pltpu.VMEM((SC_N,), jnp.float32),
        pltpu.VMEM((64,), jnp.int32),
        pltpu.VMEM((64,), jnp.float32),
        pltpu.VMEM((64,), jnp.int32),
        pltpu.VMEM((64,), jnp.float32),
        pltpu.SemaphoreType.DMA,
        pltpu.SemaphoreType.DMA,
    ),
)
def _k3(ca_hbm, zf_hbm, egf_hbm, scmax_hbm, sums_hbm, out_hbm,
        mx_v, su_v, stab_v, ca_v, zc_v, egc_v, o_v,
        ex_ca, ex_z, ex_eg, ex_o, sem, sem2):
    wid = lax.axis_index("s") * NC + lax.axis_index("c")
    s0 = wid * SC_N
    pltpu.sync_copy(ca_hbm.at[pl.ds(s0, SC_N)], ca_v)
    g1 = pltpu.async_copy(zf_hbm.at[ca_v], zc_v, sem)
    g2 = pltpu.async_copy(egf_hbm.at[ca_v], egc_v, sem2)

    pltpu.sync_copy(scmax_hbm, mx_v)
    m0 = jnp.max(mx_v[0, :])
    m1 = jnp.max(mx_v[1, :])
    m = jnp.maximum(m0, m1)
    f0 = jnp.exp(jnp.zeros((L,), jnp.float32) + (m0 - m))
    f1 = jnp.exp(jnp.zeros((L,), jnp.float32) + (m1 - m))
    pltpu.sync_copy(sums_hbm, su_v)
    for t in range(NSEG // L):
        sl = pl.ds(t * L, L)
        stab_v[sl] = su_v[0, sl] * f0 + su_v[1, sl] * f1

    g1.wait()
    g2.wait()

    def crow(r, _):
        for cj in range(4):
            sl = pl.ds(r * 64 + cj * L, L)
            s16 = plsc.load_gather(stab_v, [egc_v[sl]])
            y = jnp.exp(zc_v[sl] - m) / s16
            o_v[sl] = (1.0 - y) + y
        return 0
    lax.fori_loop(0, SW, crow, 0)
    pltpu.sync_copy(o_v, out_hbm.at[pl.ds(s0, SC_N)])

    @pl.when(wid < US_EXTRA)
    def _extra():
        e0 = (SW * NW + wid) * 64
        pltpu.sync_copy(ca_hbm.at[pl.ds(e0, 64)], ex_ca)
        pltpu.sync_copy(zf_hbm.at[ex_ca], ex_z)
        pltpu.sync_copy(egf_hbm.at[ex_ca], ex_eg)
        for j in range(4):
            sl = pl.ds(j * L, L)
            s16 = plsc.load_gather(stab_v, [ex_eg[sl]])
            y = jnp.exp(ex_z[sl] - m) / s16
            ex_o[sl] = (1.0 - y) + y
        pltpu.sync_copy(ex_o, out_hbm.at[pl.ds(e0, 64)])


def kernel(candidate_edges, loglog_u, sampled_edges, edges_logits):
    cols = jnp.concatenate([candidate_edges[:, 0], candidate_edges[:, 1]])
    ca = sampled_edges[:, 5]
    z, sums, scmax = _k12(cols, loglog_u, edges_logits)
    return _k3(ca, z, cols, scmax, sums)


# speedup vs baseline: 1.0108x; 1.0097x over previous
import functools

import jax
import jax.numpy as jnp
from jax import lax
from jax.experimental import pallas as pl
from jax.experimental.pallas import tpu as pltpu
from jax.experimental.pallas import tpu_sc as plsc

NFE = 6400000
NCAND = 1000000
NSAMP = 200000
NSEG = 1024

NC, NS, L = 2, 16, 16
NW = NC * NS

UC = NCAND // 64
UW = UC // NW
CW = UW * 64
UC_EXTRA = UC - UW * NW
US = NSAMP // 64
SW = US // NW
SC_N = SW * 64
US_EXTRA = US - SW * NW

_mesh = plsc.VectorSubcoreMesh(
    core_axis_name="c", subcore_axis_name="s",
    num_cores=NC, num_subcores=NS)

_NEG_BIG = -3.0e38

_params = pltpu.CompilerParams(
    needs_layout_passes=False, use_tc_tiling_on_sc=False)


@functools.partial(
    pl.kernel,
    out_type=(
        jax.ShapeDtypeStruct((NCAND,), jnp.float32),
        jax.ShapeDtypeStruct((NC, NSEG), jnp.float32),
        jax.ShapeDtypeStruct((NC, L), jnp.float32),
    ),
    mesh=_mesh,
    compiler_params=_params,
    scratch_types=(
        pltpu.VMEM((CW,), jnp.int32),
        pltpu.VMEM((CW,), jnp.int32),
        pltpu.VMEM((CW,), jnp.float32),
        pltpu.VMEM((CW,), jnp.float32),
        pltpu.VMEM((64,), jnp.int32),
        pltpu.VMEM((64,), jnp.int32),
        pltpu.VMEM((64,), jnp.float32),
        pltpu.VMEM((64,), jnp.float32),
        pltpu.VMEM((L,), jnp.float32),
        pltpu.VMEM((L,), jnp.float32),
        pltpu.VMEM((NS, L), jnp.float32),
        pltpu.VMEM((NSEG,), jnp.float32),
        pltpu.VMEM_SHARED((NS, L), jnp.float32),
        pltpu.VMEM_SHARED((NSEG,), jnp.float32),
        pltpu.SemaphoreType.DMA,
        pltpu.SemaphoreType.DMA,
    ),
)
def _k12(eg_hbm, ei_hbm, u_hbm, logits_hbm,
         z_hbm, sums_hbm, scmax_hbm,
         ei_v, eg_v, lg_v, u_v, ex_ei, ex_eg, ex_lg, ex_u,
         pm_v, max_ref, maxbuf_v, zeros_v, sh_max, sh_sums, sem, sem2):
    cid = lax.axis_index("c")
    sid = lax.axis_index("s")
    wid = sid * NC + cid
    c0 = wid * CW
    max_ref[:] = jnp.full((L,), _NEG_BIG, jnp.float32)

    CH = CW // 2
    pltpu.sync_copy(ei_hbm.at[pl.ds(c0, CW)], ei_v)
    ga = pltpu.async_copy(
        logits_hbm.at[ei_v.at[pl.ds(0, CH)]], lg_v.at[pl.ds(0, CH)], sem)
    gb = pltpu.async_copy(
        logits_hbm.at[ei_v.at[pl.ds(CH, CH)]], lg_v.at[pl.ds(CH, CH)], sem2)
    pltpu.sync_copy(u_hbm.at[pl.ds(c0, CW)], u_v)
    pltpu.sync_copy(eg_hbm.at[pl.ds(c0, CW)], eg_v)

    for t in range(NSEG // L):
        zeros_v[pl.ds(t * L, L)] = jnp.zeros((L,), jnp.float32)

    @pl.when(sid == 0)
    def _init():
        pltpu.sync_copy(zeros_v, sh_sums)

    @pl.when(wid < UC_EXTRA)
    def _extra_z():
        e0 = (UW * NW + wid) * 64
        pltpu.sync_copy(ei_hbm.at[pl.ds(e0, 64)], ex_ei)
        pltpu.sync_copy(u_hbm.at[pl.ds(e0, 64)], ex_u)
        pltpu.sync_copy(eg_hbm.at[pl.ds(e0, 64)], ex_eg)
        pltpu.sync_copy(logits_hbm.at[ex_ei], ex_lg)
        for j in range(4):
            sl = pl.ds(j * L, L)
            zj = ex_lg[sl] + ex_u[sl]
            ex_lg[sl] = zj
            max_ref[:] = jnp.maximum(max_ref[:], zj)
        pltpu.sync_copy(ex_lg, z_hbm.at[pl.ds(e0, 64)])

    def zrow(r, mx):
        for cj in range(16):
            sl = pl.ds(r * 256 + cj * L, L)
            zj = lg_v[sl] + u_v[sl]
            lg_v[sl] = zj
            mx = jnp.maximum(mx, zj)
        return mx
    ga.wait()
    mx = lax.fori_loop(0, CH // 256, zrow, max_ref[:])
    gb.wait()
    mx = lax.fori_loop(CH // 256, CW // 256, zrow, mx)
    pm_v[:] = mx
    zw = pltpu.async_copy(lg_v, z_hbm.at[pl.ds(c0, CW)], sem)

    pltpu.sync_copy(pm_v, sh_max.at[sid])
    plsc.subcore_barrier()
    pltpu.sync_copy(sh_max, maxbuf_v)

    def mrow(r, acc):
        return jnp.maximum(acc, maxbuf_v[r, :])
    m16 = lax.fori_loop(0, NS, mrow, jnp.full((L,), _NEG_BIG, jnp.float32))
    m = jnp.max(m16)
    zw.wait()

    def erow(r, _):
        for cj in range(16):
            sl = pl.ds(r * 256 + cj * L, L)
            lg_v[sl] = jnp.exp(lg_v[sl] - m)
        return 0
    lax.fori_loop(0, CW // 256, erow, 0)

    pltpu.async_copy(lg_v, sh_sums.at[eg_v], sem, add=True).wait()

    @pl.when(wid < UC_EXTRA)
    def _extra_sum():
        for j in range(4):
            sl = pl.ds(j * L, L)
            ex_lg[sl] = jnp.exp(ex_lg[sl] - m)
        pltpu.sync_copy(ex_lg, sh_sums.at[ex_eg], add=True)

    plsc.subcore_barrier()

    @pl.when(sid == 0)
    def _out():
        pltpu.sync_copy(sh_sums, sums_hbm.at[cid])
        pm_v[:] = jnp.zeros((L,), jnp.float32) + m
        pltpu.sync_copy(pm_v, scmax_hbm.at[cid])


@functools.partial(
    pl.kernel,
    out_type=jax.ShapeDtypeStruct((NSAMP,), jnp.float32),
    mesh=_mesh,
    compiler_params=_params,
    scratch_types=(
        pltpu.VMEM((NC, L), jnp.float32),
        pltpu.VMEM((NC, NSEG), jnp.float32),
        pltpu.VMEM((NSEG,), jnp.float32),
        pltpu.VMEM((SC_N,), jnp.int32),
        pltpu.VMEM((SC_N,), jnp.float32),
        pltpu.VMEM((SC_N,), jnp.int32),
        pltpu.VMEM((SC_N,), jnp.float32),
        pltpu.VMEM((64,), jnp.int32),
        pltpu.VMEM((64,), jnp.float32),
        pltpu.VMEM((64,), jnp.int32),
        pltpu.VMEM((64,), jnp.float32),
        pltpu.SemaphoreType.DMA,
        pltpu.SemaphoreType.DMA,
    ),
)
def _k3(ca_hbm, zf_hbm, egf_hbm, scmax_hbm, sums_hbm, out_hbm,
        mx_v, su_v, stab_v, ca_v, zc_v, egc_v, o_v,
        ex_ca, ex_z, ex_eg, ex_o, sem, sem2):
    wid = lax.axis_index("s") * NC + lax.axis_index("c")
    s0 = wid * SC_N
    pltpu.sync_copy(ca_hbm.at[pl.ds(s0, SC_N)], ca_v)
    g1 = pltpu.async_copy(zf_hbm.at[ca_v], zc_v, sem)
    g2 = pltpu.async_copy(egf_hbm.at[ca_v], egc_v, sem2)

    pltpu.sync_copy(scmax_hbm, mx_v)
    m0 = jnp.max(mx_v[0, :])
    m1 = jnp.max(mx_v[1, :])
    m = jnp.maximum(m0, m1)
    f0 = jnp.exp(jnp.zeros((L,), jnp.float32) + (m0 - m))
    f1 = jnp.exp(jnp.zeros((L,), jnp.float32) + (m1 - m))
    pltpu.sync_copy(sums_hbm, su_v)
    for t in range(NSEG // L):
        sl = pl.ds(t * L, L)
        stab_v[sl] = su_v[0, sl] * f0 + su_v[1, sl] * f1

    g1.wait()
    g2.wait()

    def crow(r, _):
        for cj in range(4):
            sl = pl.ds(r * 64 + cj * L, L)
            s16 = plsc.load_gather(stab_v, [egc_v[sl]])
            y = jnp.exp(zc_v[sl] - m) / s16
            o_v[sl] = (1.0 - y) + y
        return 0
    lax.fori_loop(0, SW, crow, 0)
    pltpu.sync_copy(o_v, out_hbm.at[pl.ds(s0, SC_N)])

    @pl.when(wid < US_EXTRA)
    def _extra():
        e0 = (SW * NW + wid) * 64
        pltpu.sync_copy(ca_hbm.at[pl.ds(e0, 64)], ex_ca)
        pltpu.sync_copy(zf_hbm.at[ex_ca], ex_z)
        pltpu.sync_copy(egf_hbm.at[ex_ca], ex_eg)
        for j in range(4):
            sl = pl.ds(j * L, L)
            s16 = plsc.load_gather(stab_v, [ex_eg[sl]])
            y = jnp.exp(ex_z[sl] - m) / s16
            ex_o[sl] = (1.0 - y) + y
        pltpu.sync_copy(ex_o, out_hbm.at[pl.ds(e0, 64)])


def kernel(candidate_edges, loglog_u, sampled_edges, edges_logits):
    eg = candidate_edges[:, 0]
    ei = candidate_edges[:, 1]
    ca = sampled_edges[:, 5]
    z, sums, scmax = _k12(eg, ei, loglog_u, edges_logits)
    return _k3(ca, z, eg, scmax, sums)


# speedup vs baseline: 1.2029x; 1.1900x over previous
import functools

import jax
import jax.numpy as jnp
from jax import lax
from jax.experimental import pallas as pl
from jax.experimental.pallas import tpu as pltpu
from jax.experimental.pallas import tpu_sc as plsc

NFE = 6400000
NCAND = 1000000
NSAMP = 200000
NSEG = 1024

NC, NS, L = 2, 16, 16
NW = NC * NS

UC = NCAND // 64
UW = UC // NW
CW = UW * 64
UC_EXTRA = UC - UW * NW
US = NSAMP // 64
SW = US // NW
SC_N = SW * 64
US_EXTRA = US - SW * NW

_mesh = plsc.VectorSubcoreMesh(
    core_axis_name="c", subcore_axis_name="s",
    num_cores=NC, num_subcores=NS)

_NEG_BIG = -3.0e38

_params = pltpu.CompilerParams(
    needs_layout_passes=False, use_tc_tiling_on_sc=False)


@functools.partial(
    pl.kernel,
    out_type=(
        jax.ShapeDtypeStruct((NCAND,), jnp.float32),
        jax.ShapeDtypeStruct((NC, NSEG), jnp.float32),
        jax.ShapeDtypeStruct((NC, L), jnp.float32),
    ),
    mesh=_mesh,
    compiler_params=_params,
    scratch_types=(
        pltpu.VMEM((CW,), jnp.int32),
        pltpu.VMEM((CW,), jnp.int32),
        pltpu.VMEM((CW,), jnp.float32),
        pltpu.VMEM((CW,), jnp.float32),
        pltpu.VMEM((64,), jnp.int32),
        pltpu.VMEM((64,), jnp.int32),
        pltpu.VMEM((64,), jnp.float32),
        pltpu.VMEM((64,), jnp.float32),
        pltpu.VMEM((L,), jnp.float32),
        pltpu.VMEM((L,), jnp.float32),
        pltpu.VMEM((NS, L), jnp.float32),
        pltpu.VMEM((NSEG,), jnp.float32),
        pltpu.VMEM_SHARED((NS, L), jnp.float32),
        pltpu.VMEM_SHARED((NSEG,), jnp.float32),
        pltpu.SemaphoreType.DMA,
        pltpu.SemaphoreType.DMA,
    ),
)
def _k12(eg_hbm, ei_hbm, u_hbm, logits_hbm,
         z_hbm, sums_hbm, scmax_hbm,
         ei_v, eg_v, lg_v, u_v, ex_ei, ex_eg, ex_lg, ex_u,
         pm_v, max_ref, maxbuf_v, zeros_v, sh_max, sh_sums, sem, sem2):
    cid = lax.axis_index("c")
    sid = lax.axis_index("s")
    wid = sid * NC + cid
    c0 = wid * CW
    max_ref[:] = jnp.full((L,), _NEG_BIG, jnp.float32)

    CH = CW // 2
    pltpu.sync_copy(ei_hbm.at[pl.ds(c0, CW)], ei_v)
    ga = pltpu.async_copy(
        logits_hbm.at[ei_v.at[pl.ds(0, CH)]], lg_v.at[pl.ds(0, CH)], sem)
    gb = pltpu.async_copy(
        logits_hbm.at[ei_v.at[pl.ds(CH, CH)]], lg_v.at[pl.ds(CH, CH)], sem2)
    pltpu.sync_copy(u_hbm.at[pl.ds(c0, CW)], u_v)
    pltpu.sync_copy(eg_hbm.at[pl.ds(c0, CW)], eg_v)

    for t in range(NSEG // L):
        zeros_v[pl.ds(t * L, L)] = jnp.zeros((L,), jnp.float32)

    @pl.when(sid == 0)
    def _init():
        pltpu.sync_copy(zeros_v, sh_sums)

    @pl.when(wid < UC_EXTRA)
    def _extra_z():
        e0 = (UW * NW + wid) * 64
        pltpu.sync_copy(ei_hbm.at[pl.ds(e0, 64)], ex_ei)
        pltpu.sync_copy(u_hbm.at[pl.ds(e0, 64)], ex_u)
        pltpu.sync_copy(eg_hbm.at[pl.ds(e0, 64)], ex_eg)
        pltpu.sync_copy(logits_hbm.at[ex_ei], ex_lg)
        for j in range(4):
            sl = pl.ds(j * L, L)
            zj = ex_lg[sl] + ex_u[sl]
            ex_lg[sl] = zj
            max_ref[:] = jnp.maximum(max_ref[:], zj)
        pltpu.sync_copy(ex_lg, z_hbm.at[pl.ds(e0, 64)])

    def zrow(r, mx):
        for cj in range(16):
            sl = pl.ds(r * 256 + cj * L, L)
            zj = lg_v[sl] + u_v[sl]
            lg_v[sl] = zj
            mx = jnp.maximum(mx, zj)
        return mx
    ga.wait()
    mx = lax.fori_loop(0, CH // 256, zrow, max_ref[:])
    gb.wait()
    mx = lax.fori_loop(CH // 256, CW // 256, zrow, mx)
    pm_v[:] = mx
    zw = pltpu.async_copy(lg_v, z_hbm.at[pl.ds(c0, CW)], sem)

    pltpu.sync_copy(pm_v, sh_max.at[sid])
    plsc.subcore_barrier()
    pltpu.sync_copy(sh_max, maxbuf_v)

    def mrow(r, acc):
        return jnp.maximum(acc, maxbuf_v[r, :])
    m16 = lax.fori_loop(0, NS, mrow, jnp.full((L,), _NEG_BIG, jnp.float32))
    m = jnp.max(m16)
    zw.wait()

    def erow(r, _):
        for cj in range(16):
            sl = pl.ds(r * 256 + cj * L, L)
            lg_v[sl] = jnp.exp(lg_v[sl] - m)
        return 0

    lax.fori_loop(0, CH // 256, erow, 0)
    sa = pltpu.async_copy(
        lg_v.at[pl.ds(0, CH)], sh_sums.at[eg_v.at[pl.ds(0, CH)]], sem,
        add=True)
    lax.fori_loop(CH // 256, CW // 256, erow, 0)
    sb = pltpu.async_copy(
        lg_v.at[pl.ds(CH, CH)], sh_sums.at[eg_v.at[pl.ds(CH, CH)]], sem2,
        add=True)
    sa.wait()
    sb.wait()

    @pl.when(wid < UC_EXTRA)
    def _extra_sum():
        for j in range(4):
            sl = pl.ds(j * L, L)
            ex_lg[sl] = jnp.exp(ex_lg[sl] - m)
        pltpu.sync_copy(ex_lg, sh_sums.at[ex_eg], add=True)

    plsc.subcore_barrier()

    @pl.when(sid == 0)
    def _out():
        pltpu.sync_copy(sh_sums, sums_hbm.at[cid])
        pm_v[:] = jnp.zeros((L,), jnp.float32) + m
        pltpu.sync_copy(pm_v, scmax_hbm.at[cid])


@functools.partial(
    pl.kernel,
    out_type=jax.ShapeDtypeStruct((NSAMP,), jnp.float32),
    mesh=_mesh,
    compiler_params=_params,
    scratch_types=(
        pltpu.VMEM((NC, L), jnp.float32),
        pltpu.VMEM((NC, NSEG), jnp.float32),
        pltpu.VMEM((NSEG,), jnp.float32),
        pltpu.VMEM((SC_N,), jnp.int32),
        pltpu.VMEM((SC_N,), jnp.float32),
        pltpu.VMEM((SC_N,), jnp.int32),
        pltpu.VMEM((SC_N,), jnp.float32),
        pltpu.VMEM((64,), jnp.int32),
        pltpu.VMEM((64,), jnp.float32),
        pltpu.VMEM((64,), jnp.int32),
        pltpu.VMEM((64,), jnp.float32),
        pltpu.SemaphoreType.DMA,
        pltpu.SemaphoreType.DMA,
    ),
)
def _k3(ca_hbm, zf_hbm, egf_hbm, scmax_hbm, sums_hbm, out_hbm,
        mx_v, su_v, stab_v, ca_v, zc_v, egc_v, o_v,
        ex_ca, ex_z, ex_eg, ex_o, sem, sem2):
    wid = lax.axis_index("s") * NC + lax.axis_index("c")
    s0 = wid * SC_N
    pltpu.sync_copy(ca_hbm.at[pl.ds(s0, SC_N)], ca_v)
    g1 = pltpu.async_copy(zf_hbm.at[ca_v], zc_v, sem)
    g2 = pltpu.async_copy(egf_hbm.at[ca_v], egc_v, sem2)

    pltpu.sync_copy(scmax_hbm, mx_v)
    m0 = jnp.max(mx_v[0, :])
    m1 = jnp.max(mx_v[1, :])
    m = jnp.maximum(m0, m1)
    f0 = jnp.exp(jnp.zeros((L,), jnp.float32) + (m0 - m))
    f1 = jnp.exp(jnp.zeros((L,), jnp.float32) + (m1 - m))
    pltpu.sync_copy(sums_hbm, su_v)
    for t in range(NSEG // L):
        sl = pl.ds(t * L, L)
        stab_v[sl] = su_v[0, sl] * f0 + su_v[1, sl] * f1

    g1.wait()
    g2.wait()

    def crow(r, _):
        for cj in range(4):
            sl = pl.ds(r * 64 + cj * L, L)
            s16 = plsc.load_gather(stab_v, [egc_v[sl]])
            y = jnp.exp(zc_v[sl] - m) / s16
            o_v[sl] = (1.0 - y) + y
        return 0
    lax.fori_loop(0, SW, crow, 0)
    pltpu.sync_copy(o_v, out_hbm.at[pl.ds(s0, SC_N)])

    @pl.when(wid < US_EXTRA)
    def _extra():
        e0 = (SW * NW + wid) * 64
        pltpu.sync_copy(ca_hbm.at[pl.ds(e0, 64)], ex_ca)
        pltpu.sync_copy(zf_hbm.at[ex_ca], ex_z)
        pltpu.sync_copy(egf_hbm.at[ex_ca], ex_eg)
        for j in range(4):
            sl = pl.ds(j * L, L)
            s16 = plsc.load_gather(stab_v, [ex_eg[sl]])
            y = jnp.exp(ex_z[sl] - m) / s16
            ex_o[sl] = (1.0 - y) + y
        pltpu.sync_copy(ex_o, out_hbm.at[pl.ds(e0, 64)])


def kernel(candidate_edges, loglog_u, sampled_edges, edges_logits):
    eg = candidate_edges[:, 0]
    ei = candidate_edges[:, 1]
    ca = sampled_edges[:, 5]
    z, sums, scmax = _k12(eg, ei, loglog_u, edges_logits)
    return _k3(ca, z, eg, scmax, sums)
